# Initial kernel scaffold; baseline (speedup 1.0000x reference)
import jax
import jax.numpy as jnp
from jax.experimental import pallas as pl


def kernel(x, edge_index, W1, b1, gamma, beta, W2, b2):
    raise NotImplementedError("write your pallas kernel here")


# speedup vs baseline: 6.6598x; 6.6598x over previous
import functools

import jax
import jax.numpy as jnp
from jax import lax
from jax.experimental import pallas as pl
from jax.experimental.pallas import tpu as pltpu
from jax.experimental.pallas import tpu_sc as plsc

N = 10000
E = 320000
F = 128
H = 128
BN_EPS = 1e-5

EC = 128
ROWS = E // EC
NW = 32
BASE_ROWS = ROWS // NW
EXTRA = ROWS - BASE_ROWS * NW
NSUB = 16
NPAD = 10240
RPS = NPAD // NSUB
LAST_VALID = N - 15 * RPS


def _make_sc_kernel():
    mesh = plsc.VectorSubcoreMesh(core_axis_name="c", subcore_axis_name="s")

    @functools.partial(
        pl.kernel,
        out_type=(
            jax.ShapeDtypeStruct((N, F), jnp.float32),
            jax.ShapeDtypeStruct((N, F), jnp.float32),
        ),
        mesh=mesh,
        scratch_types=[
            pltpu.VMEM((EC,), jnp.int32),
            pltpu.VMEM((EC,), jnp.int32),
            pltpu.VMEM((EC, F), jnp.float32),
            pltpu.VMEM_SHARED((NPAD, F), jnp.float32),
            pltpu.SemaphoreType.DMA,
        ],
    )
    def sc_aggr(src_hbm, dst_hbm, x_hbm, out0, out1, sidx, didx, rows, aggr, sem):
        c = lax.axis_index("c")
        s = lax.axis_index("s")
        w = c * NSUB + s

        def zrow(i, carry):
            for j in range(F // 16):
                rows[i, pl.ds(j * 16, 16)] = jnp.zeros((16,), jnp.float32)
            return carry

        lax.fori_loop(0, EC, zrow, 0)
        for k in range(RPS // EC):
            pltpu.sync_copy(rows, aggr.at[pl.ds(s * RPS + k * EC, EC)])
        plsc.subcore_barrier()

        start = w * BASE_ROWS + jnp.minimum(w, EXTRA)
        cnt = BASE_ROWS + jnp.where(w < EXTRA, 1, 0)

        def body(i, carry):
            r = start + i
            pltpu.sync_copy(src_hbm.at[pl.ds(r * EC, EC)], sidx)
            pltpu.sync_copy(dst_hbm.at[pl.ds(r * EC, EC)], didx)
            pltpu.async_copy(x_hbm.at[sidx], rows, sem).wait()
            pltpu.sync_copy(rows, aggr.at[didx], add=True)
            return carry

        lax.fori_loop(0, cnt, body, 0)
        plsc.subcore_barrier()

        out = [out0, out1]
        for ci in range(2):
            @pl.when(c == ci)
            def _(ci=ci):
                @pl.when(s < NSUB - 1)
                def _():
                    pltpu.sync_copy(
                        aggr.at[pl.ds(s * RPS, RPS)],
                        out[ci].at[pl.ds(s * RPS, RPS)],
                    )

                @pl.when(s == NSUB - 1)
                def _():
                    pltpu.sync_copy(
                        aggr.at[pl.ds((NSUB - 1) * RPS, LAST_VALID)],
                        out[ci].at[pl.ds((NSUB - 1) * RPS, LAST_VALID)],
                    )

    return sc_aggr


_sc_aggr = _make_sc_kernel()

BLKR = 1000
NB = N // BLKR


def _tc_body(x_ref, p0_ref, p1_ref, w1_ref, w2_ref, prm_ref, out_ref, h1s, stat):
    p = pl.program_id(0)
    b = pl.program_id(1)

    @pl.when(p == 0)
    def _():
        a = x_ref[...] + p0_ref[...] + p1_ref[...]
        h1 = jnp.maximum(
            jnp.dot(a, w1_ref[...], preferred_element_type=jnp.float32)
            + prm_ref[0:1, :],
            0.0,
        )
        h1s[pl.ds(b * BLKR, BLKR), :] = h1
        s1 = jnp.sum(h1, axis=0, keepdims=True)
        s2 = jnp.sum(h1 * h1, axis=0, keepdims=True)

        @pl.when(b == 0)
        def _():
            stat[0:1, :] = s1
            stat[1:2, :] = s2

        @pl.when(b > 0)
        def _():
            stat[0:1, :] = stat[0:1, :] + s1
            stat[1:2, :] = stat[1:2, :] + s2

    @pl.when(p == 1)
    def _():
        @pl.when(b == 0)
        def _():
            mean = stat[0:1, :] * (1.0 / N)
            var = stat[1:2, :] * (1.0 / N) - mean * mean
            rstd = lax.rsqrt(var + BN_EPS)
            scale = prm_ref[1:2, :] * rstd
            stat[2:3, :] = scale
            stat[3:4, :] = prm_ref[2:3, :] - mean * scale

        h1 = h1s[pl.ds(b * BLKR, BLKR), :]
        h2 = h1 * stat[2:3, :] + stat[3:4, :]
        out_ref[...] = (
            jnp.dot(h2, w2_ref[...], preferred_element_type=jnp.float32)
            + prm_ref[3:4, :]
        )


def _tc_mlp(x, p0, p1, W1, W2, prm):
    return pl.pallas_call(
        _tc_body,
        grid=(2, NB),
        in_specs=[
            pl.BlockSpec((BLKR, F), lambda p, b: (jnp.where(p == 0, b, 0), 0)),
            pl.BlockSpec((BLKR, F), lambda p, b: (jnp.where(p == 0, b, 0), 0)),
            pl.BlockSpec((BLKR, F), lambda p, b: (jnp.where(p == 0, b, 0), 0)),
            pl.BlockSpec((F, H), lambda p, b: (0, 0)),
            pl.BlockSpec((H, H), lambda p, b: (0, 0)),
            pl.BlockSpec((4, H), lambda p, b: (0, 0)),
        ],
        out_specs=pl.BlockSpec((BLKR, H), lambda p, b: (jnp.where(p == 0, 0, b), 0)),
        out_shape=jax.ShapeDtypeStruct((N, H), jnp.float32),
        scratch_shapes=[
            pltpu.VMEM((N, H), jnp.float32),
            pltpu.VMEM((8, 128), jnp.float32),
        ],
    )(x, p0, p1, W1, W2, prm)


def kernel(x, edge_index, W1, b1, gamma, beta, W2, b2):
    src = edge_index[0]
    dst = edge_index[1]
    p0, p1 = _sc_aggr(src, dst, x)
    prm = jnp.stack([b1, gamma, beta, b2])
    return _tc_mlp(x, p0, p1, W1, W2, prm)
